# Initial kernel scaffold; baseline (speedup 1.0000x reference)
import jax
import jax.numpy as jnp
from jax.experimental import pallas as pl


def kernel(x, edge_index, batch, emb, W_l1, W_r1, b1, p1, W_l2, W_r2, b2, p2, W_l3, W_r3, b3, p3, lin1_W, lin1_b, lin2_W, lin2_b, lin3_W, lin3_b):
    raise NotImplementedError("write your pallas kernel here")


# speedup vs baseline: 3.1058x; 3.1058x over previous
import functools
import jax
import jax.numpy as jnp
from jax import lax
from jax.experimental import pallas as pl
from jax.experimental.pallas import tpu as pltpu
from jax.experimental.pallas import tpu_sc as plsc

NC, NS, L = 2, 16, 16
NW = NC * NS
NG = 512
N_PAD = 10240
E_PAD = 327680
D = 128
DX = 144
BN = 256
NB = N_PAD // BN
FMIN = float(jnp.finfo(jnp.float32).min)

def _make_sc_kernels():
    mesh = plsc.VectorSubcoreMesh(core_axis_name="c", subcore_axis_name="s",
                                  num_cores=NC)

    @functools.partial(
        pl.kernel, mesh=mesh,
        out_type=jax.ShapeDtypeStruct((N_PAD, D), jnp.float32),
        scratch_types=[
            pltpu.VMEM((5, 64), jnp.int32),
            pltpu.VMEM((320, D), jnp.float32),
            pltpu.SemaphoreType.DMA,
        ],
    )
    def emb_gather(emb_hbm, idx_hbm, out_hbm, idx_v, rows_v, sem):
        wid = lax.axis_index("s") * NC + lax.axis_index("c")
        base = wid * 320
        pltpu.sync_copy(idx_hbm.at[wid], idx_v)

        def chunk(j, _):
            pltpu.async_copy(emb_hbm.at[idx_v.at[j]],
                             rows_v.at[pl.ds(j * 64, 64)], sem).wait()
            return _

        lax.fori_loop(0, 5, chunk, None)
        pltpu.sync_copy(rows_v, out_hbm.at[pl.ds(base, 320)])

    @functools.partial(
        pl.kernel, mesh=mesh,
        compiler_params=pltpu.CompilerParams(needs_layout_passes=False),
        out_type=jax.ShapeDtypeStruct((NC, N_PAD, D), jnp.float32),
        scratch_types=[
            pltpu.VMEM((160, 64), jnp.int32),
            pltpu.VMEM((160, 64), jnp.int32),
            pltpu.VMEM((64, D), jnp.float32),
            pltpu.VMEM_SHARED((N_PAD, D), jnp.float32),
            pltpu.SemaphoreType.DMA,
        ],
    )
    def edge_agg(x_hbm, src3_hbm, dst3_hbm, zeros_hbm, agg_out,
                 src_v, dst_v, rows_v, agg_sh, sem):
        c = lax.axis_index("c")
        s = lax.axis_index("s")
        pltpu.sync_copy(src3_hbm.at[s * NC + c], src_v)
        pltpu.sync_copy(dst3_hbm.at[s * NC + c], dst_v)
        pltpu.sync_copy(zeros_hbm, rows_v)

        def z(i, _):
            pltpu.sync_copy(rows_v, agg_sh.at[pl.ds(s * 640 + i * 64, 64)])
            return _

        lax.fori_loop(0, 10, z, None)
        plsc.subcore_barrier()

        def chunk(j, _):
            pltpu.async_copy(x_hbm.at[src_v.at[j]], rows_v, sem).wait()
            pltpu.sync_copy(rows_v, agg_sh.at[dst_v.at[j]], add=True)
            return _

        lax.fori_loop(0, 160, chunk, None)
        plsc.subcore_barrier()

        def w(i, _):
            pltpu.sync_copy(agg_sh.at[pl.ds(s * 640 + i * 64, 64)], rows_v)
            pltpu.sync_copy(rows_v,
                            agg_out.at[c, pl.ds(s * 640 + i * 64, 64)])
            return _

        lax.fori_loop(0, 10, w, None)

    EPW = E_PAD // NW

    @functools.partial(
        pl.kernel, mesh=mesh,
        compiler_params=pltpu.CompilerParams(needs_layout_passes=False),
        out_type=jax.ShapeDtypeStruct((NW, N_PAD), jnp.float32),
        scratch_types=[
            pltpu.VMEM((EPW,), jnp.int32),
            pltpu.VMEM((EPW,), jnp.int32),
            pltpu.VMEM((N_PAD,), jnp.float32),
            pltpu.VMEM((N_PAD,), jnp.float32),
        ],
    )
    def deg_part(srcf_hbm, dstf_hbm, nm_hbm, deg_out,
                 srcf_v, dstf_v, nm_v, deg_v):
        wid = lax.axis_index("s") * NC + lax.axis_index("c")
        pltpu.sync_copy(srcf_hbm.at[wid], srcf_v)
        pltpu.sync_copy(dstf_hbm.at[wid], dstf_v)
        pltpu.sync_copy(nm_hbm, nm_v)

        def zd(k, _):
            deg_v[pl.ds(k * 16, 16)] = jnp.zeros((16,), jnp.float32)
            return _

        lax.fori_loop(0, N_PAD // 16, zd, None)

        def dloop(k, _):
            sv = srcf_v[pl.ds(k * 16, 16)]
            dv = dstf_v[pl.ds(k * 16, 16)]
            vals = plsc.load_gather(nm_v, [sv])
            plsc.addupdate_scatter(deg_v, [dv], vals)
            return _

        lax.fori_loop(0, EPW // 16, dloop, None)
        pltpu.sync_copy(deg_v, deg_out.at[wid])

    return emb_gather, edge_agg, deg_part


_SC_CACHE = []


def _emb_gather(emb, idx):
    if not _SC_CACHE:
        _SC_CACHE.extend(_make_sc_kernels())
    return _SC_CACHE[0](emb, idx)


def _edge_agg(x_tab, src3, dst3, zeros):
    if not _SC_CACHE:
        _SC_CACHE.extend(_make_sc_kernels())
    return _SC_CACHE[1](x_tab, src3, dst3, zeros)


def _deg_part(srcf, dstf, nm):
    if not _SC_CACHE:
        _SC_CACHE.extend(_make_sc_kernels())
    return _SC_CACHE[2](srcf, dstf, nm)


def _dense_body(a0, a1, dall, ones, x, nm, wl, wr, b, p, h_out, s_out):
    feat = a0[...] + a1[...]
    deg = lax.dot_general(dall[...], ones[...], (((0,), (0,)), ((), ())),
                          preferred_element_type=jnp.float32)
    deg = jnp.clip(deg, 1.0, None)
    h = lax.dot_general(feat / deg, wl[...], (((1,), (1,)), ((), ())),
                        preferred_element_type=jnp.float32)
    h = h + lax.dot_general(x[...], wr[...], (((1,), (1,)), ((), ())),
                            preferred_element_type=jnp.float32)
    h = jax.nn.relu(h + b[...]) * nm[...]
    pv = p[...]
    nrm = jnp.sqrt(jnp.sum(pv * pv))
    h_out[...] = h
    s_out[...] = lax.dot_general(h, pv, (((1,), (0,)), ((), ())),
                                 preferred_element_type=jnp.float32) / nrm


def _sage_dense(a0, a1, dall, ones, x, nm_col, wl, wr, b, p):
    return pl.pallas_call(
        _dense_body,
        grid=(NB,),
        in_specs=[
            pl.BlockSpec((BN, D), lambda i: (i, 0)),
            pl.BlockSpec((BN, D), lambda i: (i, 0)),
            pl.BlockSpec((NW, BN), lambda i: (0, i)),
            pl.BlockSpec((NW, 1), lambda i: (0, 0)),
            pl.BlockSpec((BN, D), lambda i: (i, 0)),
            pl.BlockSpec((BN, 1), lambda i: (i, 0)),
            pl.BlockSpec((D, D), lambda i: (0, 0)),
            pl.BlockSpec((D, D), lambda i: (0, 0)),
            pl.BlockSpec((1, D), lambda i: (0, 0)),
            pl.BlockSpec((D, 1), lambda i: (0, 0)),
        ],
        out_specs=[
            pl.BlockSpec((BN, D), lambda i: (i, 0)),
            pl.BlockSpec((BN, 1), lambda i: (i, 0)),
        ],
        out_shape=[
            jax.ShapeDtypeStruct((N_PAD, D), jnp.float32),
            jax.ShapeDtypeStruct((N_PAD, 1), jnp.float32),
        ],
    )(a0, a1, dall, ones, x, nm_col, wl, wr, b, p)


def _rank_body(s_col, nm_col, b_col, s_row, nm_row, b_row, keep_out,
               cnt_acc, alive_acc):
    i = pl.program_id(0)
    j = pl.program_id(1)

    @pl.when(j == 0)
    def _():
        cnt_acc[...] = jnp.zeros_like(cnt_acc)
        alive_acc[...] = jnp.zeros_like(alive_acc)

    sr_i = jnp.where(nm_col[...] > 0, s_col[...], FMIN)
    sr_j = jnp.where(nm_row[...] > 0, s_row[...], FMIN)
    ig = lax.broadcasted_iota(jnp.int32, (BN, 1), 0) + i * BN
    jg = lax.broadcasted_iota(jnp.int32, (1, BN), 1) + j * BN
    same = b_row[...] == b_col[...]
    higher = (sr_j > sr_i) | ((sr_j == sr_i) & (jg < ig))
    cnt_acc[...] += jnp.sum(jnp.where(same & higher, 1.0, 0.0), axis=1,
                            keepdims=True)
    alive_acc[...] += jnp.sum(jnp.where(same, nm_row[...], 0.0), axis=1,
                              keepdims=True)

    @pl.when(j == pl.num_programs(1) - 1)
    def _():
        k = jnp.ceil(0.8 * alive_acc[...])
        keep_out[...] = jnp.where((cnt_acc[...] < k) & (nm_col[...] > 0),
                                  1.0, 0.0)


def _topk_keep(s_col, nm_col, b_col, s_row, nm_row, b_row):
    return pl.pallas_call(
        _rank_body,
        grid=(NB, NB),
        in_specs=[
            pl.BlockSpec((BN, 1), lambda i, j: (i, 0)),
            pl.BlockSpec((BN, 1), lambda i, j: (i, 0)),
            pl.BlockSpec((BN, 1), lambda i, j: (i, 0)),
            pl.BlockSpec((1, BN), lambda i, j: (0, j)),
            pl.BlockSpec((1, BN), lambda i, j: (0, j)),
            pl.BlockSpec((1, BN), lambda i, j: (0, j)),
        ],
        out_specs=pl.BlockSpec((BN, 1), lambda i, j: (i, 0)),
        out_shape=jax.ShapeDtypeStruct((N_PAD, 1), jnp.float32),
        scratch_shapes=[
            pltpu.VMEM((BN, 1), jnp.float32),
            pltpu.VMEM((BN, 1), jnp.float32),
        ],
    )(s_col, nm_col, b_col, s_row, nm_row, b_row)


def _pool_body(h, s_col, keep_col, b_row, b_col, xp_out, sum_out, max_out,
               cnt_out):
    i = pl.program_id(0)

    @pl.when(i == 0)
    def _():
        sum_out[...] = jnp.zeros_like(sum_out)
        cnt_out[...] = jnp.zeros_like(cnt_out)
        max_out[...] = jnp.full_like(max_out, FMIN)

    keep = keep_col[...]
    xp = h[...] * jnp.tanh(s_col[...]) * keep
    xp_out[...] = xp
    gid = lax.broadcasted_iota(jnp.int32, (NG, 1), 0)
    onehot = jnp.where(gid == b_row[...], 1.0, 0.0)
    sum_out[...] += lax.dot_general(onehot, xp, (((1,), (0,)), ((), ())),
                                    preferred_element_type=jnp.float32)
    cnt_out[...] += lax.dot_general(onehot, keep, (((1,), (0,)), ((), ())),
                                    preferred_element_type=jnp.float32)
    xm = jnp.where(keep > 0, xp, FMIN)
    bc = b_col[...]
    g_lo = b_col[0, 0]
    g_hi = jnp.minimum(b_col[BN - 1, 0], NG - 1)

    def gloop(g, _):
        masked = jnp.where(bc == g, xm, FMIN)
        mx = jnp.max(masked, axis=0, keepdims=True)
        cur = max_out[pl.ds(g, 1), :]
        max_out[pl.ds(g, 1), :] = jnp.maximum(cur, mx)
        return _

    lax.fori_loop(g_lo, g_hi + 1, gloop, None)


def _pool_readout(h, s_col, keep_col, b_row, b_col):
    return pl.pallas_call(
        _pool_body,
        grid=(NB,),
        in_specs=[
            pl.BlockSpec((BN, D), lambda i: (i, 0)),
            pl.BlockSpec((BN, 1), lambda i: (i, 0)),
            pl.BlockSpec((BN, 1), lambda i: (i, 0)),
            pl.BlockSpec((1, BN), lambda i: (0, i)),
            pl.BlockSpec((BN, 1), lambda i: (i, 0)),
        ],
        out_specs=[
            pl.BlockSpec((BN, D), lambda i: (i, 0)),
            pl.BlockSpec((NG, D), lambda i: (0, 0)),
            pl.BlockSpec((NG, D), lambda i: (0, 0)),
            pl.BlockSpec((NG, 1), lambda i: (0, 0)),
        ],
        out_shape=[
            jax.ShapeDtypeStruct((N_PAD, D), jnp.float32),
            jax.ShapeDtypeStruct((NG, D), jnp.float32),
            jax.ShapeDtypeStruct((NG, D), jnp.float32),
            jax.ShapeDtypeStruct((NG, 1), jnp.float32),
        ],
    )(h, s_col, keep_col, b_row, b_col)


def _mlp_body(s1, m1, c1, s2, m2, c2, s3, m3, c3, w1, b1, w2, b2, w3, b3,
              out):
    def ro(sm, mx, cn):
        cnt = cn[...]
        mean = sm[...] / jnp.clip(cnt, 1.0, None)
        mxv = jnp.where(cnt > 0, mx[...], 0.0)
        return jnp.concatenate([mxv, mean], axis=1)

    z = ro(s1, m1, c1) + ro(s2, m2, c2) + ro(s3, m3, c3)
    z = jax.nn.relu(lax.dot_general(z, w1[...], (((1,), (1,)), ((), ())),
                                    preferred_element_type=jnp.float32)
                    + b1[...])
    z = jax.nn.relu(lax.dot_general(z, w2[...], (((1,), (1,)), ((), ())),
                                    preferred_element_type=jnp.float32)
                    + b2[...])
    z = jnp.sum(z * w3[...], axis=1, keepdims=True) + b3[0, 0]
    out[...] = 1.0 / (1.0 + jnp.exp(-z))


def _final_mlp(readouts, w1, b1, w2, b2, w3, b3):
    args = []
    for (sm, mx, cn) in readouts:
        args += [sm, mx, cn]
    args += [w1, b1.reshape(1, -1), w2, b2.reshape(1, -1),
             w3, b3.reshape(1, -1)]
    return pl.pallas_call(
        _mlp_body,
        out_shape=jax.ShapeDtypeStruct((NG, 1), jnp.float32),
    )(*args)


def kernel(x, edge_index, batch, emb, W_l1, W_r1, b1, p1, W_l2, W_r2, b2, p2,
           W_l3, W_r3, b3, p3, lin1_W, lin1_b, lin2_W, lin2_b,
           lin3_W, lin3_b):
    N = x.shape[0]
    E = edge_index.shape[1]

    idx = jnp.pad(x[:, 0].astype(jnp.int32), (0, N_PAD - N)).reshape(NW, 5, 64)
    srcf = jnp.pad(edge_index[0].astype(jnp.int32), (0, E_PAD - E),
                   constant_values=N).reshape(NW, E_PAD // NW)
    dstf = jnp.pad(edge_index[1].astype(jnp.int32), (0, E_PAD - E),
                   constant_values=N).reshape(NW, E_PAD // NW)
    src3 = srcf.reshape(NW, 160, 64)
    dst3 = dstf.reshape(NW, 160, 64)
    b_pad = jnp.pad(batch.astype(jnp.int32), (0, N_PAD - N),
                    constant_values=NG)
    b_col = b_pad.reshape(N_PAD, 1)
    b_row = b_pad.reshape(1, N_PAD)
    zeros = jnp.zeros((64, D), jnp.float32)
    ones32 = jnp.ones((NW, 1), jnp.float32)

    h = _emb_gather(emb, idx)
    nm = jnp.pad(jnp.ones((N,), jnp.float32), (0, N_PAD - N))

    params = [(W_l1, W_r1, b1, p1), (W_l2, W_r2, b2, p2), (W_l3, W_r3, b3, p3)]
    readouts = []
    for (wl, wr, bb, pp) in params:
        agg2 = _edge_agg(h, src3, dst3, zeros)
        dall = _deg_part(srcf, dstf, nm)
        nm_col = nm.reshape(N_PAD, 1)
        hd, s_col = _sage_dense(agg2[0], agg2[1], dall, ones32, h, nm_col,
                                wl, wr, bb.reshape(1, D), pp.reshape(D, 1))
        keep_col = _topk_keep(s_col, nm_col, b_col,
                              s_col.reshape(1, N_PAD),
                              nm.reshape(1, N_PAD), b_row)
        h, sm, mx, cn = _pool_readout(hd, s_col, keep_col, b_row, b_col)
        nm = keep_col[:, 0]
        readouts.append((sm, mx, cn))

    out = _final_mlp(readouts, lin1_W, lin1_b, lin2_W, lin2_b, lin3_W, lin3_b)
    return out[:, 0]


# speedup vs baseline: 3.2917x; 1.0599x over previous
import functools
import jax
import jax.numpy as jnp
from jax import lax
from jax.experimental import pallas as pl
from jax.experimental.pallas import tpu as pltpu
from jax.experimental.pallas import tpu_sc as plsc

NC, NS, L = 2, 16, 16
NW = NC * NS
NG = 512
N_PAD = 10240
E_PAD = 327680
D = 128
DX = 144
BN = 256
NB = N_PAD // BN
FMIN = float(jnp.finfo(jnp.float32).min)

def _make_sc_kernels():
    mesh = plsc.VectorSubcoreMesh(core_axis_name="c", subcore_axis_name="s",
                                  num_cores=NC)

    @functools.partial(
        pl.kernel, mesh=mesh,
        out_type=jax.ShapeDtypeStruct((N_PAD, D), jnp.float32),
        scratch_types=[
            pltpu.VMEM((5, 64), jnp.int32),
            pltpu.VMEM((320, D), jnp.float32),
            pltpu.SemaphoreType.DMA,
        ],
    )
    def emb_gather(emb_hbm, idx_hbm, out_hbm, idx_v, rows_v, sem):
        wid = lax.axis_index("s") * NC + lax.axis_index("c")
        base = wid * 320
        pltpu.sync_copy(idx_hbm.at[wid], idx_v)

        def chunk(j, _):
            pltpu.async_copy(emb_hbm.at[idx_v.at[j]],
                             rows_v.at[pl.ds(j * 64, 64)], sem).wait()
            return _

        lax.fori_loop(0, 5, chunk, None)
        pltpu.sync_copy(rows_v, out_hbm.at[pl.ds(base, 320)])

    @functools.partial(
        pl.kernel, mesh=mesh,
        compiler_params=pltpu.CompilerParams(needs_layout_passes=False),
        out_type=jax.ShapeDtypeStruct((NC, N_PAD, D), jnp.float32),
        scratch_types=[
            pltpu.VMEM((16, 128), jnp.int32),
            pltpu.VMEM((16, 128), jnp.int32),
            pltpu.VMEM((128, D), jnp.float32),
            pltpu.VMEM((128, D), jnp.float32),
            pltpu.VMEM_SHARED((N_PAD, D), jnp.float32),
            pltpu.SemaphoreType.DMA,
            pltpu.SemaphoreType.DMA,
        ],
    )
    def edge_agg(x_hbm, src3_hbm, dst3_hbm, zeros_hbm, agg_out,
                 sidx, didx, rows_a, rows_b, agg_sh, sem_a, sem_b):
        c = lax.axis_index("c")
        s = lax.axis_index("s")
        wid = s * NC + c
        pltpu.sync_copy(zeros_hbm, rows_a)

        def z(i, _):
            pltpu.sync_copy(rows_a, agg_sh.at[pl.ds(s * 640 + i * 128, 128)])
            return _

        lax.fori_loop(0, 5, z, None)
        plsc.subcore_barrier()

        dummy = x_hbm.at[pl.ds(0, 128)]

        def outer(o, _):
            pltpu.sync_copy(src3_hbm.at[wid, pl.ds(o * 16, 16)], sidx)
            pltpu.sync_copy(dst3_hbm.at[wid, pl.ds(o * 16, 16)], didx)
            pltpu.async_copy(x_hbm.at[sidx.at[0]], rows_a, sem_a)

            def inner(j2, __):
                pltpu.make_async_copy(dummy, rows_a, sem_a).wait()
                pltpu.async_copy(x_hbm.at[sidx.at[2 * j2 + 1]], rows_b,
                                 sem_b)
                pltpu.sync_copy(rows_a, agg_sh.at[didx.at[2 * j2]],
                                add=True)
                pltpu.make_async_copy(dummy, rows_b, sem_b).wait()
                nxt = jnp.minimum(2 * j2 + 2, 15)
                pltpu.async_copy(x_hbm.at[sidx.at[nxt]], rows_a, sem_a)
                pltpu.sync_copy(rows_b, agg_sh.at[didx.at[2 * j2 + 1]],
                                add=True)
                return __

            lax.fori_loop(0, 8, inner, None)
            pltpu.make_async_copy(dummy, rows_a, sem_a).wait()
            return _

        lax.fori_loop(0, 5, outer, None)
        plsc.subcore_barrier()

        def w(i, _):
            pltpu.sync_copy(agg_sh.at[pl.ds(s * 640 + i * 128, 128)], rows_a)
            pltpu.sync_copy(rows_a,
                            agg_out.at[c, pl.ds(s * 640 + i * 128, 128)])
            return _

        lax.fori_loop(0, 5, w, None)

    EPW = E_PAD // NW

    @functools.partial(
        pl.kernel, mesh=mesh,
        compiler_params=pltpu.CompilerParams(needs_layout_passes=False),
        out_type=jax.ShapeDtypeStruct((NW, N_PAD), jnp.float32),
        scratch_types=[
            pltpu.VMEM((EPW,), jnp.int32),
            pltpu.VMEM((EPW,), jnp.int32),
            pltpu.VMEM((N_PAD,), jnp.float32),
            pltpu.VMEM((N_PAD,), jnp.float32),
        ],
    )
    def deg_part(srcf_hbm, dstf_hbm, nm_hbm, deg_out,
                 srcf_v, dstf_v, nm_v, deg_v):
        wid = lax.axis_index("s") * NC + lax.axis_index("c")
        pltpu.sync_copy(srcf_hbm.at[wid], srcf_v)
        pltpu.sync_copy(dstf_hbm.at[wid], dstf_v)
        pltpu.sync_copy(nm_hbm, nm_v)

        def zd(k, _):
            deg_v[pl.ds(k * 16, 16)] = jnp.zeros((16,), jnp.float32)
            return _

        lax.fori_loop(0, N_PAD // 16, zd, None)

        def dloop(k, _):
            sv = srcf_v[pl.ds(k * 16, 16)]
            dv = dstf_v[pl.ds(k * 16, 16)]
            vals = plsc.load_gather(nm_v, [sv])
            plsc.addupdate_scatter(deg_v, [dv], vals)
            return _

        lax.fori_loop(0, EPW // 16, dloop, None)
        pltpu.sync_copy(deg_v, deg_out.at[wid])

    return emb_gather, edge_agg, deg_part


_SC_CACHE = []


def _emb_gather(emb, idx):
    if not _SC_CACHE:
        _SC_CACHE.extend(_make_sc_kernels())
    return _SC_CACHE[0](emb, idx)


def _edge_agg(x_tab, src3, dst3, zeros):
    if not _SC_CACHE:
        _SC_CACHE.extend(_make_sc_kernels())
    return _SC_CACHE[1](x_tab, src3, dst3, zeros)


def _deg_part(srcf, dstf, nm):
    if not _SC_CACHE:
        _SC_CACHE.extend(_make_sc_kernels())
    return _SC_CACHE[2](srcf, dstf, nm)


def _dense_body(a0, a1, dall, ones, x, nm, wl, wr, b, p, h_out, s_out):
    feat = a0[...] + a1[...]
    deg = lax.dot_general(dall[...], ones[...], (((0,), (0,)), ((), ())),
                          preferred_element_type=jnp.float32)
    deg = jnp.clip(deg, 1.0, None)
    h = lax.dot_general(feat / deg, wl[...], (((1,), (1,)), ((), ())),
                        preferred_element_type=jnp.float32)
    h = h + lax.dot_general(x[...], wr[...], (((1,), (1,)), ((), ())),
                            preferred_element_type=jnp.float32)
    h = jax.nn.relu(h + b[...]) * nm[...]
    pv = p[...]
    nrm = jnp.sqrt(jnp.sum(pv * pv))
    h_out[...] = h
    s_out[...] = lax.dot_general(h, pv, (((1,), (0,)), ((), ())),
                                 preferred_element_type=jnp.float32) / nrm


def _sage_dense(a0, a1, dall, ones, x, nm_col, wl, wr, b, p):
    return pl.pallas_call(
        _dense_body,
        grid=(NB,),
        in_specs=[
            pl.BlockSpec((BN, D), lambda i: (i, 0)),
            pl.BlockSpec((BN, D), lambda i: (i, 0)),
            pl.BlockSpec((NW, BN), lambda i: (0, i)),
            pl.BlockSpec((NW, 1), lambda i: (0, 0)),
            pl.BlockSpec((BN, D), lambda i: (i, 0)),
            pl.BlockSpec((BN, 1), lambda i: (i, 0)),
            pl.BlockSpec((D, D), lambda i: (0, 0)),
            pl.BlockSpec((D, D), lambda i: (0, 0)),
            pl.BlockSpec((1, D), lambda i: (0, 0)),
            pl.BlockSpec((D, 1), lambda i: (0, 0)),
        ],
        out_specs=[
            pl.BlockSpec((BN, D), lambda i: (i, 0)),
            pl.BlockSpec((BN, 1), lambda i: (i, 0)),
        ],
        out_shape=[
            jax.ShapeDtypeStruct((N_PAD, D), jnp.float32),
            jax.ShapeDtypeStruct((N_PAD, 1), jnp.float32),
        ],
    )(a0, a1, dall, ones, x, nm_col, wl, wr, b, p)


def _rank_body(s_col, nm_col, b_col, s_row, nm_row, b_row, keep_out,
               cnt_acc, alive_acc):
    i = pl.program_id(0)
    j = pl.program_id(1)

    @pl.when(j == 0)
    def _():
        cnt_acc[...] = jnp.zeros_like(cnt_acc)
        alive_acc[...] = jnp.zeros_like(alive_acc)

    sr_i = jnp.where(nm_col[...] > 0, s_col[...], FMIN)
    sr_j = jnp.where(nm_row[...] > 0, s_row[...], FMIN)
    ig = lax.broadcasted_iota(jnp.int32, (BN, 1), 0) + i * BN
    jg = lax.broadcasted_iota(jnp.int32, (1, BN), 1) + j * BN
    same = b_row[...] == b_col[...]
    higher = (sr_j > sr_i) | ((sr_j == sr_i) & (jg < ig))
    cnt_acc[...] += jnp.sum(jnp.where(same & higher, 1.0, 0.0), axis=1,
                            keepdims=True)
    alive_acc[...] += jnp.sum(jnp.where(same, nm_row[...], 0.0), axis=1,
                              keepdims=True)

    @pl.when(j == pl.num_programs(1) - 1)
    def _():
        k = jnp.ceil(0.8 * alive_acc[...])
        keep_out[...] = jnp.where((cnt_acc[...] < k) & (nm_col[...] > 0),
                                  1.0, 0.0)


def _topk_keep(s_col, nm_col, b_col, s_row, nm_row, b_row):
    return pl.pallas_call(
        _rank_body,
        grid=(NB, NB),
        in_specs=[
            pl.BlockSpec((BN, 1), lambda i, j: (i, 0)),
            pl.BlockSpec((BN, 1), lambda i, j: (i, 0)),
            pl.BlockSpec((BN, 1), lambda i, j: (i, 0)),
            pl.BlockSpec((1, BN), lambda i, j: (0, j)),
            pl.BlockSpec((1, BN), lambda i, j: (0, j)),
            pl.BlockSpec((1, BN), lambda i, j: (0, j)),
        ],
        out_specs=pl.BlockSpec((BN, 1), lambda i, j: (i, 0)),
        out_shape=jax.ShapeDtypeStruct((N_PAD, 1), jnp.float32),
        scratch_shapes=[
            pltpu.VMEM((BN, 1), jnp.float32),
            pltpu.VMEM((BN, 1), jnp.float32),
        ],
    )(s_col, nm_col, b_col, s_row, nm_row, b_row)


def _pool_body(h, s_col, keep_col, b_row, b_col, xp_out, sum_out, max_out,
               cnt_out):
    i = pl.program_id(0)

    @pl.when(i == 0)
    def _():
        sum_out[...] = jnp.zeros_like(sum_out)
        cnt_out[...] = jnp.zeros_like(cnt_out)
        max_out[...] = jnp.full_like(max_out, FMIN)

    keep = keep_col[...]
    xp = h[...] * jnp.tanh(s_col[...]) * keep
    xp_out[...] = xp
    gid = lax.broadcasted_iota(jnp.int32, (NG, 1), 0)
    onehot = jnp.where(gid == b_row[...], 1.0, 0.0)
    sum_out[...] += lax.dot_general(onehot, xp, (((1,), (0,)), ((), ())),
                                    preferred_element_type=jnp.float32)
    cnt_out[...] += lax.dot_general(onehot, keep, (((1,), (0,)), ((), ())),
                                    preferred_element_type=jnp.float32)
    xm = jnp.where(keep > 0, xp, FMIN)
    bc = b_col[...]
    g_lo = b_col[0, 0]
    g_hi = jnp.minimum(b_col[BN - 1, 0], NG - 1)

    def gloop(g, _):
        masked = jnp.where(bc == g, xm, FMIN)
        mx = jnp.max(masked, axis=0, keepdims=True)
        cur = max_out[pl.ds(g, 1), :]
        max_out[pl.ds(g, 1), :] = jnp.maximum(cur, mx)
        return _

    lax.fori_loop(g_lo, g_hi + 1, gloop, None)


def _pool_readout(h, s_col, keep_col, b_row, b_col):
    return pl.pallas_call(
        _pool_body,
        grid=(NB,),
        in_specs=[
            pl.BlockSpec((BN, D), lambda i: (i, 0)),
            pl.BlockSpec((BN, 1), lambda i: (i, 0)),
            pl.BlockSpec((BN, 1), lambda i: (i, 0)),
            pl.BlockSpec((1, BN), lambda i: (0, i)),
            pl.BlockSpec((BN, 1), lambda i: (i, 0)),
        ],
        out_specs=[
            pl.BlockSpec((BN, D), lambda i: (i, 0)),
            pl.BlockSpec((NG, D), lambda i: (0, 0)),
            pl.BlockSpec((NG, D), lambda i: (0, 0)),
            pl.BlockSpec((NG, 1), lambda i: (0, 0)),
        ],
        out_shape=[
            jax.ShapeDtypeStruct((N_PAD, D), jnp.float32),
            jax.ShapeDtypeStruct((NG, D), jnp.float32),
            jax.ShapeDtypeStruct((NG, D), jnp.float32),
            jax.ShapeDtypeStruct((NG, 1), jnp.float32),
        ],
    )(h, s_col, keep_col, b_row, b_col)


def _mlp_body(s1, m1, c1, s2, m2, c2, s3, m3, c3, w1, b1, w2, b2, w3, b3,
              out):
    def ro(sm, mx, cn):
        cnt = cn[...]
        mean = sm[...] / jnp.clip(cnt, 1.0, None)
        mxv = jnp.where(cnt > 0, mx[...], 0.0)
        return jnp.concatenate([mxv, mean], axis=1)

    z = ro(s1, m1, c1) + ro(s2, m2, c2) + ro(s3, m3, c3)
    z = jax.nn.relu(lax.dot_general(z, w1[...], (((1,), (1,)), ((), ())),
                                    preferred_element_type=jnp.float32)
                    + b1[...])
    z = jax.nn.relu(lax.dot_general(z, w2[...], (((1,), (1,)), ((), ())),
                                    preferred_element_type=jnp.float32)
                    + b2[...])
    z = jnp.sum(z * w3[...], axis=1, keepdims=True) + b3[0, 0]
    out[...] = 1.0 / (1.0 + jnp.exp(-z))


def _final_mlp(readouts, w1, b1, w2, b2, w3, b3):
    args = []
    for (sm, mx, cn) in readouts:
        args += [sm, mx, cn]
    args += [w1, b1.reshape(1, -1), w2, b2.reshape(1, -1),
             w3, b3.reshape(1, -1)]
    return pl.pallas_call(
        _mlp_body,
        out_shape=jax.ShapeDtypeStruct((NG, 1), jnp.float32),
    )(*args)


def kernel(x, edge_index, batch, emb, W_l1, W_r1, b1, p1, W_l2, W_r2, b2, p2,
           W_l3, W_r3, b3, p3, lin1_W, lin1_b, lin2_W, lin2_b,
           lin3_W, lin3_b):
    N = x.shape[0]
    E = edge_index.shape[1]

    idx = jnp.pad(x[:, 0].astype(jnp.int32), (0, N_PAD - N)).reshape(NW, 5, 64)
    srcf = jnp.pad(edge_index[0].astype(jnp.int32), (0, E_PAD - E),
                   constant_values=N).reshape(NW, E_PAD // NW)
    dstf = jnp.pad(edge_index[1].astype(jnp.int32), (0, E_PAD - E),
                   constant_values=N).reshape(NW, E_PAD // NW)
    src3 = srcf.reshape(NW, 80, 128)
    dst3 = dstf.reshape(NW, 80, 128)
    b_pad = jnp.pad(batch.astype(jnp.int32), (0, N_PAD - N),
                    constant_values=NG)
    b_col = b_pad.reshape(N_PAD, 1)
    b_row = b_pad.reshape(1, N_PAD)
    zeros = jnp.zeros((128, D), jnp.float32)
    ones32 = jnp.ones((NW, 1), jnp.float32)

    h = _emb_gather(emb, idx)
    nm = jnp.pad(jnp.ones((N,), jnp.float32), (0, N_PAD - N))

    params = [(W_l1, W_r1, b1, p1), (W_l2, W_r2, b2, p2), (W_l3, W_r3, b3, p3)]
    readouts = []
    for (wl, wr, bb, pp) in params:
        agg2 = _edge_agg(h, src3, dst3, zeros)
        dall = _deg_part(srcf, dstf, nm)
        nm_col = nm.reshape(N_PAD, 1)
        hd, s_col = _sage_dense(agg2[0], agg2[1], dall, ones32, h, nm_col,
                                wl, wr, bb.reshape(1, D), pp.reshape(D, 1))
        keep_col = _topk_keep(s_col, nm_col, b_col,
                              s_col.reshape(1, N_PAD),
                              nm.reshape(1, N_PAD), b_row)
        h, sm, mx, cn = _pool_readout(hd, s_col, keep_col, b_row, b_col)
        nm = keep_col[:, 0]
        readouts.append((sm, mx, cn))

    out = _final_mlp(readouts, lin1_W, lin1_b, lin2_W, lin2_b, lin3_W, lin3_b)
    return out[:, 0]
